# Initial kernel scaffold; baseline (speedup 1.0000x reference)
import jax
import jax.numpy as jnp
from jax.experimental import pallas as pl


def kernel(x, pos_edge_index, neg_edge_index, W_pos_l, W_pos_r, b_pos, W_neg_l, W_neg_r, b_neg, gamma, beta):
    raise NotImplementedError("write your pallas kernel here")


# speedup vs baseline: 8.9245x; 8.9245x over previous
import functools

import jax
import jax.numpy as jnp
from jax import lax
from jax.experimental import pallas as pl
from jax.experimental.pallas import tpu as pltpu
from jax.experimental.pallas import tpu_sc as plsc

N_NODES = 10000
N_EDGES = 320000
IN_DIMS = 128
OUT_DIMS = 64
EPS = 1e-5

D = 80
NTILES = 16
CHUNK = 128
NCHUNK = 157
PER_TILE = CHUNK * NCHUNK
E_PAD = PER_TILE * NTILES
ROWS_PT = 632
ROWS = ROWS_PT * NTILES
DUMMY_DST = N_NODES


def _pre_body(x_ref, wl_ref, yp_ref, yn_ref):
    xw = jnp.dot(x_ref[...], wl_ref[...], preferred_element_type=jnp.float32)
    r = x_ref.shape[0]
    col = lax.broadcasted_iota(jnp.int32, (r, D - OUT_DIMS), 1)
    tail = jnp.where(col == 0, 1.0, 0.0).astype(jnp.float32)
    yp_ref[...] = jnp.concatenate([xw[:, :OUT_DIMS], tail], axis=1)
    yn_ref[...] = jnp.concatenate([xw[:, OUT_DIMS:], tail], axis=1)


def _pre(x, wl_cat):
    return pl.pallas_call(
        _pre_body,
        out_shape=[
            jax.ShapeDtypeStruct((N_NODES, D), jnp.float32),
            jax.ShapeDtypeStruct((N_NODES, D), jnp.float32),
        ],
        grid=(5,),
        in_specs=[
            pl.BlockSpec((N_NODES // 5, IN_DIMS), lambda i: (i, 0)),
            pl.BlockSpec((IN_DIMS, 2 * OUT_DIMS), lambda i: (0, 0)),
        ],
        out_specs=[
            pl.BlockSpec((N_NODES // 5, D), lambda i: (i, 0)),
            pl.BlockSpec((N_NODES // 5, D), lambda i: (i, 0)),
        ],
    )(x, wl_cat)


def _sc_body(yp, yn, psrc, pdst, nsrc, ndst, zeros_hbm, outp, outn,
             src_v, dst_v, buf, acc, sem):
    c = lax.axis_index("c")
    s = lax.axis_index("s")

    pltpu.sync_copy(zeros_hbm.at[pl.ds(s * ROWS_PT, ROWS_PT)],
                    acc.at[pl.ds(s * ROWS_PT, ROWS_PT)])
    plsc.subcore_barrier()

    def run(y_h, src_h, dst_h, out_h):
        pltpu.sync_copy(src_h.at[s], src_v)
        pltpu.sync_copy(dst_h.at[s], dst_v)

        def body(j, carry):
            pltpu.async_copy(y_h.at[src_v.at[j]], buf, sem).wait()
            pltpu.sync_copy(buf, acc.at[dst_v.at[j]], add=True)
            return carry

        lax.fori_loop(0, NCHUNK, body, 0)
        plsc.subcore_barrier()
        pltpu.sync_copy(acc.at[pl.ds(s * ROWS_PT, ROWS_PT)],
                        out_h.at[pl.ds(s * ROWS_PT, ROWS_PT)])

    @pl.when(c == 0)
    def _():
        run(yp, psrc, pdst, outp)

    @pl.when(c == 1)
    def _():
        run(yn, nsrc, ndst, outn)


_sc_agg = functools.partial(
    pl.kernel,
    _sc_body,
    out_type=[
        jax.ShapeDtypeStruct((ROWS, D), jnp.float32),
        jax.ShapeDtypeStruct((ROWS, D), jnp.float32),
    ],
    mesh=plsc.VectorSubcoreMesh(core_axis_name="c", subcore_axis_name="s"),
    compiler_params=pltpu.CompilerParams(use_tc_tiling_on_sc=False),
    scratch_types=[
        pltpu.VMEM((NCHUNK, CHUNK), jnp.int32),
        pltpu.VMEM((NCHUNK, CHUNK), jnp.int32),
        pltpu.VMEM((CHUNK, D), jnp.float32),
        pltpu.VMEM_SHARED((ROWS, D), jnp.float32),
        pltpu.SemaphoreType.DMA,
    ],
)()


def _post_body(x_ref, sp_ref, sn_ref, wr_ref, b_ref, g_ref, be_ref, out_ref):
    xr = jnp.dot(x_ref[...], wr_ref[...], preferred_element_type=jnp.float32)
    sp = sp_ref[...]
    sn = sn_ref[...]
    aggp = sp[:, :OUT_DIMS] / jnp.maximum(sp[:, OUT_DIMS:OUT_DIMS + 1], 1.0)
    aggn = sn[:, :OUT_DIMS] / jnp.maximum(sn[:, OUT_DIMS:OUT_DIMS + 1], 1.0)
    pre = jnp.concatenate([aggp, aggn], axis=1) + xr + b_ref[...]
    mu = jnp.mean(pre, axis=0, keepdims=True)
    var = jnp.mean(jnp.square(pre - mu), axis=0, keepdims=True)
    out = (pre - mu) * lax.rsqrt(var + EPS) * g_ref[...] + be_ref[...]
    out_ref[...] = jnp.maximum(out, 0.0)


def _post(x, sp, sn, wr_cat, b_cat, g_cat, be_cat):
    return pl.pallas_call(
        _post_body,
        out_shape=jax.ShapeDtypeStruct((N_NODES, 2 * OUT_DIMS), jnp.float32),
    )(x, sp, sn, wr_cat, b_cat, g_cat, be_cat)


def _prep_edges(edge_index):
    src = edge_index[0].astype(jnp.int32)
    dst = edge_index[1].astype(jnp.int32)
    pad = E_PAD - N_EDGES
    src = jnp.concatenate([src, jnp.zeros((pad,), jnp.int32)])
    dst = jnp.concatenate([dst, jnp.full((pad,), DUMMY_DST, jnp.int32)])
    return (src.reshape(NTILES, NCHUNK, CHUNK),
            dst.reshape(NTILES, NCHUNK, CHUNK))


def kernel(x, pos_edge_index, neg_edge_index, W_pos_l, W_pos_r, b_pos,
           W_neg_l, W_neg_r, b_neg, gamma, beta):
    psrc, pdst = _prep_edges(pos_edge_index)
    nsrc, ndst = _prep_edges(neg_edge_index)
    wl_cat = jnp.concatenate([W_pos_l, W_neg_l], axis=1)
    wr_cat = jnp.concatenate([W_pos_r, W_neg_r], axis=1)
    b_cat = jnp.concatenate([b_pos, b_neg]).reshape(1, 2 * OUT_DIMS)
    g_cat = gamma.reshape(1, 2 * OUT_DIMS)
    be_cat = beta.reshape(1, 2 * OUT_DIMS)
    zeros_hbm = jnp.zeros((ROWS, D), jnp.float32)

    yp, yn = _pre(x, wl_cat)
    sp_full, sn_full = _sc_agg(yp, yn, psrc, pdst, nsrc, ndst, zeros_hbm)
    sp = sp_full[:N_NODES]
    sn = sn_full[:N_NODES]
    return _post(x, sp, sn, wr_cat, b_cat, g_cat, be_cat)
